# baseline (speedup 1.0000x reference)
import jax
import jax.numpy as jnp
from jax.experimental import pallas as pl

_LAYERS = 3
_K = 1024
_D = 64
_BLK = 1024


def _rowsum64(s):
    acc = s[:, 0:8]
    for k in range(1, s.shape[1] // 8):
        acc = acc + s[:, 8 * k:8 * k + 8]
    a = acc[:, :4] + acc[:, 4:8]
    a = a[:, :2] + a[:, 2:4]
    return a[:, 0:1] + a[:, 1:2]


def _rvq_body(z_ref, cb_ref, q_ref, idx_ref):
    residual = z_ref[...]
    b = residual.shape[0]
    iota = jax.lax.broadcasted_iota(jnp.int32, (b, _K), 1)
    quant = jnp.zeros_like(residual)
    for l in range(_LAYERS):
        cb = cb_ref[l]
        r2 = _rowsum64(residual * residual)
        c2 = jnp.sum(cb * cb, axis=1)[None, :]
        dot = jax.lax.dot_general(residual, cb, (((1,), (1,)), ((), ())),
                                  preferred_element_type=jnp.float32)
        d2 = r2 + c2 - 2.0 * dot
        dist = jnp.sqrt(jnp.maximum(d2, 0.0))
        m = jnp.min(dist, axis=1, keepdims=True)
        idx = jnp.min(jnp.where(dist == m, iota, _K), axis=1)
        onehot = (iota == idx[:, None]).astype(jnp.bfloat16)
        cb_hi = cb.astype(jnp.bfloat16)
        rem = cb - cb_hi.astype(jnp.float32)
        cb_mid = rem.astype(jnp.bfloat16)
        cb_lo = (rem - cb_mid.astype(jnp.float32)).astype(jnp.bfloat16)
        dn = (((1,), (0,)), ((), ()))
        cw = ((jax.lax.dot_general(onehot, cb_hi, dn, preferred_element_type=jnp.float32)
               + jax.lax.dot_general(onehot, cb_mid, dn, preferred_element_type=jnp.float32))
              + jax.lax.dot_general(onehot, cb_lo, dn, preferred_element_type=jnp.float32))
        residual = residual - cw
        quant = quant + cw
        idx_ref[l, :] = idx
    q_ref[...] = quant


def kernel(z, codebooks):
    n, d = z.shape
    grid = (n // _BLK,)
    q, idx = pl.pallas_call(
        _rvq_body,
        grid=grid,
        in_specs=[
            pl.BlockSpec((_BLK, d), lambda i: (i, 0)),
            pl.BlockSpec((_LAYERS, _K, d), lambda i: (0, 0, 0)),
        ],
        out_specs=[
            pl.BlockSpec((_BLK, d), lambda i: (i, 0)),
            pl.BlockSpec((_LAYERS, _BLK), lambda i: (0, i)),
        ],
        out_shape=[
            jax.ShapeDtypeStruct((n, d), jnp.float32),
            jax.ShapeDtypeStruct((_LAYERS, n), jnp.int32),
        ],
    )(z, codebooks)
    return (q, idx)


# speedup vs baseline: 1.1421x; 1.1421x over previous
import jax
import jax.numpy as jnp
from jax.experimental import pallas as pl

_LAYERS = 3
_K = 1024
_D = 64
_BLK = 1024


def _rowsum64(s):
    acc = s[:, 0:8]
    for k in range(1, s.shape[1] // 8):
        acc = acc + s[:, 8 * k:8 * k + 8]
    a = acc[:, :4] + acc[:, 4:8]
    a = a[:, :2] + a[:, 2:4]
    return a[:, 0:1] + a[:, 1:2]


def _rvq_body(z_ref, cb_ref, cbh_ref, cbm_ref, cbl_ref, q_ref, idx_ref):
    residual = z_ref[...]
    b = residual.shape[0]
    iota_f = jax.lax.broadcasted_iota(jnp.int32, (b, _K), 1).astype(jnp.float32)
    quant = jnp.zeros_like(residual)
    dn = (((1,), (0,)), ((), ()))
    for l in range(_LAYERS):
        cb = cb_ref[l]
        r2 = _rowsum64(residual * residual)
        c2 = jnp.sum(cb * cb, axis=1)[None, :]
        dot = jax.lax.dot_general(residual, cb, (((1,), (1,)), ((), ())),
                                  preferred_element_type=jnp.float32)
        d2 = r2 + c2 - 2.0 * dot
        dist = jnp.sqrt(jnp.maximum(d2, 0.0))
        m = jnp.min(dist, axis=1, keepdims=True)
        idx_f = jnp.min(jnp.where(dist == m, iota_f, 2048.0),
                        axis=1, keepdims=True)
        onehot = (iota_f == idx_f).astype(jnp.bfloat16)
        cw = ((jax.lax.dot_general(onehot, cbh_ref[l], dn, preferred_element_type=jnp.float32)
               + jax.lax.dot_general(onehot, cbm_ref[l], dn, preferred_element_type=jnp.float32))
              + jax.lax.dot_general(onehot, cbl_ref[l], dn, preferred_element_type=jnp.float32))
        residual = residual - cw
        quant = quant + cw
        idx_ref[l, :] = idx_f[:, 0].astype(jnp.int32)
    q_ref[...] = quant


def kernel(z, codebooks):
    n, d = z.shape
    cbh = codebooks.astype(jnp.bfloat16)
    rem = codebooks - cbh.astype(jnp.float32)
    cbm = rem.astype(jnp.bfloat16)
    cbl = (rem - cbm.astype(jnp.float32)).astype(jnp.bfloat16)
    grid = (n // _BLK,)
    q, idx = pl.pallas_call(
        _rvq_body,
        grid=grid,
        in_specs=[
            pl.BlockSpec((_BLK, d), lambda i: (i, 0)),
            pl.BlockSpec((_LAYERS, _K, d), lambda i: (0, 0, 0)),
            pl.BlockSpec((_LAYERS, _K, d), lambda i: (0, 0, 0)),
            pl.BlockSpec((_LAYERS, _K, d), lambda i: (0, 0, 0)),
            pl.BlockSpec((_LAYERS, _K, d), lambda i: (0, 0, 0)),
        ],
        out_specs=[
            pl.BlockSpec((_BLK, d), lambda i: (i, 0)),
            pl.BlockSpec((_LAYERS, _BLK), lambda i: (0, i)),
        ],
        out_shape=[
            jax.ShapeDtypeStruct((n, d), jnp.float32),
            jax.ShapeDtypeStruct((_LAYERS, n), jnp.int32),
        ],
    )(z, codebooks, cbh, cbm, cbl)
    return (q, idx)
